# baseline (speedup 1.0000x reference)
import functools

import jax
import jax.numpy as jnp
import numpy as np
from jax import lax
from jax.experimental import pallas as pl
from jax.experimental.pallas import tpu as pltpu
from jax.experimental.pallas import tpu_sc as plsc

D = 512

_INFO = plsc.get_sparse_core_info()
NC = _INFO.num_cores
NS = _INFO.num_subcores
NW = NC * NS

CHUNK = 64

_SEG = np.arange(32).reshape(2, 16).T.reshape(-1)
_PI = (np.arange(0, D, 32)[:, None] + _SEG[None, :]).reshape(-1)


def _gather_body(n_chunks, offset, ids_hbm, table_hbm, out_hbm,
                 idx_v, rows_v, pk_v, sem_g, sem_s):
    b_per_w = n_chunks * CHUNK
    wid = lax.axis_index("s") * NC + lax.axis_index("c")
    base = wid * b_per_w
    pltpu.sync_copy(ids_hbm.at[pl.ds(offset + base, b_per_w)], idx_v)

    def start_gather(c):
        return pltpu.async_copy(
            table_hbm.at[idx_v.at[pl.ds(c * CHUNK, CHUNK)]],
            rows_v.at[c % 2],
            sem_g,
        )

    def pack_chunk(c):
        rows = rows_v.at[c % 2]
        pk = pk_v.at[c % 2]

        def row_fn(r, carry):
            for s in range(D // 32):
                a = rows[r, pl.ds(s * 32, 16)]
                b = rows[r, pl.ds(s * 32 + 16, 16)]
                ia = lax.bitcast_convert_type(a, jnp.int32)
                ib = lax.bitcast_convert_type(b, jnp.int32)
                ia = ia + 0x7FFF + (lax.shift_right_logical(ia, 16) & 1)
                ib = ib + 0x7FFF + (lax.shift_right_logical(ib, 16) & 1)
                lo = lax.shift_right_logical(ia, 16)
                hi = ib & jnp.int32(-65536)
                pk[r, pl.ds(s * 16, 16)] = lax.bitcast_convert_type(lo | hi, jnp.float32)
            return carry

        lax.fori_loop(0, CHUNK, row_fn, 0)

    gat = {0: start_gather(0)}
    sca = {}
    for c in range(n_chunks):
        gat.pop(c).wait()
        if c + 1 < n_chunks:
            gat[c + 1] = start_gather(c + 1)
        if c >= 2:
            sca.pop(c - 2).wait()
        pack_chunk(c)
        sca[c] = pltpu.async_copy(
            pk_v.at[c % 2],
            out_hbm.at[pl.ds(base + c * CHUNK, CHUNK)],
            sem_s,
        )
    for c in sorted(sca):
        sca.pop(c).wait()


def _sc_gather(ids, table, offset, n_tok):
    b_per_w = n_tok // NW
    n_chunks = b_per_w // CHUNK
    mesh = plsc.VectorSubcoreMesh(core_axis_name="c", subcore_axis_name="s")
    k = pl.kernel(
        functools.partial(_gather_body, n_chunks, offset),
        out_type=jax.ShapeDtypeStruct((n_tok, D // 2), jnp.float32),
        mesh=mesh,
        scratch_types=[
            pltpu.VMEM((n_chunks * CHUNK,), jnp.int32),
            pltpu.VMEM((2, CHUNK, D), jnp.float32),
            pltpu.VMEM((2, CHUNK, D // 2), jnp.float32),
            pltpu.SemaphoreType.DMA,
            pltpu.SemaphoreType.DMA,
        ],
    )
    return k(ids, table)


def _mlp_body(x_ref, w1_ref, b1_ref, w2_ref, b2_ref, o_ref):
    x = x_ref[...].astype(jnp.float32)
    h = jnp.dot(x, w1_ref[...], preferred_element_type=jnp.float32)
    h = jnp.maximum(h + b1_ref[...], 0.0)
    o = jnp.dot(h, w2_ref[...], preferred_element_type=jnp.float32)
    o_ref[...] = o + b2_ref[...]


def _mlp_body_alias(x_ref, w1_ref, b1_ref, w2_ref, b2_ref, prev_ref, o_ref):
    del prev_ref
    _mlp_body(x_ref, w1_ref, b1_ref, w2_ref, b2_ref, o_ref)


_WSPECS = [
    pl.BlockSpec((D, D), lambda i: (0, 0)),
    pl.BlockSpec((1, D), lambda i: (0, 0)),
    pl.BlockSpec((D, D), lambda i: (0, 0)),
    pl.BlockSpec((1, D), lambda i: (0, 0)),
]


def _tc_mlp_part(x, w1, b1, w2, b2, n_tok, off_blk, prev, block_t):
    nblk = x.shape[0] // block_t
    x_spec = pl.BlockSpec((block_t, D), lambda i: (i, 0))
    out_spec = pl.BlockSpec((block_t, D), lambda i: (i + off_blk, 0))
    if prev is None:
        return pl.pallas_call(
            _mlp_body,
            grid=(nblk,),
            in_specs=[x_spec] + _WSPECS,
            out_specs=out_spec,
            out_shape=jax.ShapeDtypeStruct((n_tok, D), jnp.float32),
        )(x, w1, b1.reshape(1, D), w2, b2.reshape(1, D))
    return pl.pallas_call(
        _mlp_body_alias,
        grid=(nblk,),
        in_specs=[x_spec] + _WSPECS + [pl.BlockSpec(memory_space=pl.ANY)],
        out_specs=out_spec,
        out_shape=jax.ShapeDtypeStruct((n_tok, D), jnp.float32),
        input_output_aliases={5: 0},
    )(x, w1, b1.reshape(1, D), w2, b2.reshape(1, D), prev)


CHUNK_SIZES = (16384, 16384)
BLOCK_T = 4096


def kernel(input_ids, emb_table, W1, b1, W2, b2):
    B, S = input_ids.shape
    ids = input_ids.reshape(-1).astype(jnp.int32)
    n_tok = ids.shape[0]
    w1p = W1[jnp.asarray(_PI), :]
    offs = [sum(CHUNK_SIZES[:k]) for k in range(len(CHUNK_SIZES))]
    xs = []
    for k, ct in enumerate(CHUNK_SIZES):
        xb = _sc_gather(ids, emb_table, offs[k], ct)
        xb = jax.lax.bitcast_convert_type(xb, jnp.bfloat16)
        xs.append(xb.reshape(ct, D))
    out = None
    for k, ct in enumerate(CHUNK_SIZES):
        out = _tc_mlp_part(
            xs[k], w1p, b1, W2, b2, n_tok,
            off_blk=offs[k] // BLOCK_T, prev=out, block_t=BLOCK_T,
        )
    return out.reshape(B, S, D)


# speedup vs baseline: 3.5776x; 3.5776x over previous
import functools

import jax
import jax.numpy as jnp
from jax import lax
from jax.experimental import pallas as pl
from jax.experimental.pallas import tpu as pltpu
from jax.experimental.pallas import tpu_sc as plsc

D = 512

_INFO = plsc.get_sparse_core_info()
NC = _INFO.num_cores
NS = _INFO.num_subcores
NW = NC * NS

CHUNK = 64


def _gather_body(n_chunks, offset, ids_hbm, table_hbm, out_hbm, idx_v, rows_v, sem_g, sem_s):
    b_per_w = n_chunks * CHUNK
    wid = lax.axis_index("s") * NC + lax.axis_index("c")
    base = wid * b_per_w
    pltpu.sync_copy(ids_hbm.at[pl.ds(offset + base, b_per_w)], idx_v)

    def start_gather(c):
        return pltpu.async_copy(
            table_hbm.at[idx_v.at[pl.ds(c * CHUNK, CHUNK)]],
            rows_v.at[c % 2],
            sem_g,
        )

    gat = {0: start_gather(0)}
    sca = {}
    for c in range(n_chunks):
        gat.pop(c).wait()
        if c >= 1:
            sca.pop(c - 1).wait()
        if c + 1 < n_chunks:
            gat[c + 1] = start_gather(c + 1)
        sca[c] = pltpu.async_copy(
            rows_v.at[c % 2],
            out_hbm.at[pl.ds(base + c * CHUNK, CHUNK)],
            sem_s,
        )
    sca.pop(n_chunks - 1).wait()


def _sc_gather(ids, table, offset, n_tok):
    b_per_w = n_tok // NW
    n_chunks = b_per_w // CHUNK
    mesh = plsc.VectorSubcoreMesh(core_axis_name="c", subcore_axis_name="s")
    k = pl.kernel(
        functools.partial(_gather_body, n_chunks, offset),
        out_type=jax.ShapeDtypeStruct((n_tok, D), jnp.float32),
        mesh=mesh,
        scratch_types=[
            pltpu.VMEM((n_chunks * CHUNK,), jnp.int32),
            pltpu.VMEM((2, CHUNK, D), jnp.float32),
            pltpu.SemaphoreType.DMA,
            pltpu.SemaphoreType.DMA,
        ],
    )
    return k(ids, table)


def _mlp_body(x_ref, w1_ref, b1_ref, w2_ref, b2_ref, o_ref):
    x = x_ref[...]
    h = jnp.dot(x, w1_ref[...], preferred_element_type=jnp.float32)
    h = jnp.maximum(h + b1_ref[...], 0.0)
    o = jnp.dot(h, w2_ref[...], preferred_element_type=jnp.float32)
    o_ref[...] = o + b2_ref[...]


def _mlp_body_alias(x_ref, w1_ref, b1_ref, w2_ref, b2_ref, prev_ref, o_ref):
    del prev_ref
    _mlp_body(x_ref, w1_ref, b1_ref, w2_ref, b2_ref, o_ref)


_WSPECS = [
    pl.BlockSpec((D, D), lambda i: (0, 0)),
    pl.BlockSpec((1, D), lambda i: (0, 0)),
    pl.BlockSpec((D, D), lambda i: (0, 0)),
    pl.BlockSpec((1, D), lambda i: (0, 0)),
]


def _tc_mlp_part(x, w1, b1, w2, b2, n_tok, off_blk, prev, block_t=2048):
    nblk = x.shape[0] // block_t
    x_spec = pl.BlockSpec((block_t, D), lambda i: (i, 0))
    out_spec = pl.BlockSpec((block_t, D), lambda i: (i + off_blk, 0))
    if prev is None:
        return pl.pallas_call(
            _mlp_body,
            grid=(nblk,),
            in_specs=[x_spec] + _WSPECS,
            out_specs=out_spec,
            out_shape=jax.ShapeDtypeStruct((n_tok, D), jnp.float32),
        )(x, w1, b1.reshape(1, D), w2, b2.reshape(1, D))
    return pl.pallas_call(
        _mlp_body_alias,
        grid=(nblk,),
        in_specs=[x_spec] + _WSPECS + [pl.BlockSpec(memory_space=pl.ANY)],
        out_specs=out_spec,
        out_shape=jax.ShapeDtypeStruct((n_tok, D), jnp.float32),
        input_output_aliases={5: 0},
    )(x, w1, b1.reshape(1, D), w2, b2.reshape(1, D), prev)


CHUNK_SIZES = (8192, 16384, 8192)
BLOCK_T = 4096


def kernel(input_ids, emb_table, W1, b1, W2, b2):
    B, S = input_ids.shape
    ids = input_ids.reshape(-1).astype(jnp.int32)
    n_tok = ids.shape[0]
    offs = [sum(CHUNK_SIZES[:k]) for k in range(len(CHUNK_SIZES))]
    xs = [
        _sc_gather(ids, emb_table, offs[k], ct)
        for k, ct in enumerate(CHUNK_SIZES)
    ]
    out = None
    for k, ct in enumerate(CHUNK_SIZES):
        out = _tc_mlp_part(
            xs[k], W1, b1, W2, b2, n_tok,
            off_blk=offs[k] // BLOCK_T, prev=out, block_t=BLOCK_T,
        )
    return out.reshape(B, S, D)


# speedup vs baseline: 3.7237x; 1.0408x over previous
import functools

import jax
import jax.numpy as jnp
from jax import lax
from jax.experimental import pallas as pl
from jax.experimental.pallas import tpu as pltpu
from jax.experimental.pallas import tpu_sc as plsc

D = 512

_INFO = plsc.get_sparse_core_info()
NC = _INFO.num_cores
NS = _INFO.num_subcores
NW = NC * NS

CHUNK = 64


def _gather_body(n_chunks, offset, ids_hbm, table_hbm, out_hbm, idx_v, rows_v, sem_g, sem_s):
    b_per_w = n_chunks * CHUNK
    wid = lax.axis_index("s") * NC + lax.axis_index("c")
    base = wid * b_per_w
    pltpu.sync_copy(ids_hbm.at[pl.ds(offset + base, b_per_w)], idx_v)

    def start_gather(c):
        return pltpu.async_copy(
            table_hbm.at[idx_v.at[pl.ds(c * CHUNK, CHUNK)]],
            rows_v.at[c % 2],
            sem_g,
        )

    gat = {0: start_gather(0)}
    sca = {}
    for c in range(n_chunks):
        gat.pop(c).wait()
        if c >= 1:
            sca.pop(c - 1).wait()
        if c + 1 < n_chunks:
            gat[c + 1] = start_gather(c + 1)
        sca[c] = pltpu.async_copy(
            rows_v.at[c % 2],
            out_hbm.at[pl.ds(base + c * CHUNK, CHUNK)],
            sem_s,
        )
    sca.pop(n_chunks - 1).wait()


def _sc_gather(ids, table, offset, n_tok):
    b_per_w = n_tok // NW
    n_chunks = b_per_w // CHUNK
    mesh = plsc.VectorSubcoreMesh(core_axis_name="c", subcore_axis_name="s")
    k = pl.kernel(
        functools.partial(_gather_body, n_chunks, offset),
        out_type=jax.ShapeDtypeStruct((n_tok, D), jnp.float32),
        mesh=mesh,
        scratch_types=[
            pltpu.VMEM((n_chunks * CHUNK,), jnp.int32),
            pltpu.VMEM((2, CHUNK, D), jnp.float32),
            pltpu.SemaphoreType.DMA,
            pltpu.SemaphoreType.DMA,
        ],
    )
    return k(ids, table)


def _mlp_body(x_ref, w1_ref, b1_ref, w2_ref, b2_ref, o_ref):
    x = x_ref[...]
    h = jnp.dot(x, w1_ref[...], preferred_element_type=jnp.float32)
    h = jnp.maximum(h + b1_ref[...], 0.0)
    o = jnp.dot(h, w2_ref[...], preferred_element_type=jnp.float32)
    o_ref[...] = o + b2_ref[...]


def _mlp_body_alias(x_ref, w1_ref, b1_ref, w2_ref, b2_ref, prev_ref, o_ref):
    del prev_ref
    _mlp_body(x_ref, w1_ref, b1_ref, w2_ref, b2_ref, o_ref)


_WSPECS = [
    pl.BlockSpec((D, D), lambda i: (0, 0)),
    pl.BlockSpec((1, D), lambda i: (0, 0)),
    pl.BlockSpec((D, D), lambda i: (0, 0)),
    pl.BlockSpec((1, D), lambda i: (0, 0)),
]


def _tc_mlp_part(x, w1, b1, w2, b2, n_tok, off_blk, prev, block_t=2048):
    nblk = x.shape[0] // block_t
    x_spec = pl.BlockSpec((block_t, D), lambda i: (i, 0))
    out_spec = pl.BlockSpec((block_t, D), lambda i: (i + off_blk, 0))
    if prev is None:
        return pl.pallas_call(
            _mlp_body,
            grid=(nblk,),
            in_specs=[x_spec] + _WSPECS,
            out_specs=out_spec,
            out_shape=jax.ShapeDtypeStruct((n_tok, D), jnp.float32),
        )(x, w1, b1.reshape(1, D), w2, b2.reshape(1, D))
    return pl.pallas_call(
        _mlp_body_alias,
        grid=(nblk,),
        in_specs=[x_spec] + _WSPECS + [pl.BlockSpec(memory_space=pl.ANY)],
        out_specs=out_spec,
        out_shape=jax.ShapeDtypeStruct((n_tok, D), jnp.float32),
        input_output_aliases={5: 0},
    )(x, w1, b1.reshape(1, D), w2, b2.reshape(1, D), prev)


CHUNK_SIZES = (16384, 16384)
BLOCK_T = 4096


def kernel(input_ids, emb_table, W1, b1, W2, b2):
    B, S = input_ids.shape
    ids = input_ids.reshape(-1).astype(jnp.int32)
    n_tok = ids.shape[0]
    offs = [sum(CHUNK_SIZES[:k]) for k in range(len(CHUNK_SIZES))]
    xs = [
        _sc_gather(ids, emb_table, offs[k], ct)
        for k, ct in enumerate(CHUNK_SIZES)
    ]
    out = None
    for k, ct in enumerate(CHUNK_SIZES):
        out = _tc_mlp_part(
            xs[k], W1, b1, W2, b2, n_tok,
            off_blk=offs[k] // BLOCK_T, prev=out, block_t=BLOCK_T,
        )
    return out.reshape(B, S, D)


# speedup vs baseline: 3.8488x; 1.0336x over previous
import functools

import jax
import jax.numpy as jnp
from jax import lax
from jax.experimental import pallas as pl
from jax.experimental.pallas import tpu as pltpu
from jax.experimental.pallas import tpu_sc as plsc

D = 512

_INFO = plsc.get_sparse_core_info()
NC = _INFO.num_cores
NS = _INFO.num_subcores
NW = NC * NS

CHUNK = 64


def _gather_body(n_chunks, offset, ids_hbm, table_hbm, out_hbm, idx_v, rows_v, sem_g, sem_s):
    b_per_w = n_chunks * CHUNK
    wid = lax.axis_index("s") * NC + lax.axis_index("c")
    base = wid * b_per_w
    pltpu.sync_copy(ids_hbm.at[pl.ds(offset + base, b_per_w)], idx_v)

    def start_gather(c):
        return pltpu.async_copy(
            table_hbm.at[idx_v.at[pl.ds(c * CHUNK, CHUNK)]],
            rows_v.at[c % 3],
            sem_g,
        )

    gat = {0: start_gather(0), 1: start_gather(1)}
    sca = {}
    for c in range(n_chunks):
        gat.pop(c).wait()
        if c >= 1:
            sca.pop(c - 1).wait()
        if c + 2 < n_chunks:
            gat[c + 2] = start_gather(c + 2)
        sca[c] = pltpu.async_copy(
            rows_v.at[c % 3],
            out_hbm.at[pl.ds(base + c * CHUNK, CHUNK)],
            sem_s,
        )
    sca.pop(n_chunks - 1).wait()


def _sc_gather(ids, table, offset, n_tok):
    b_per_w = n_tok // NW
    n_chunks = b_per_w // CHUNK
    mesh = plsc.VectorSubcoreMesh(core_axis_name="c", subcore_axis_name="s")
    k = pl.kernel(
        functools.partial(_gather_body, n_chunks, offset),
        out_type=jax.ShapeDtypeStruct((n_tok, D), jnp.float32),
        mesh=mesh,
        scratch_types=[
            pltpu.VMEM((n_chunks * CHUNK,), jnp.int32),
            pltpu.VMEM((3, CHUNK, D), jnp.float32),
            pltpu.SemaphoreType.DMA,
            pltpu.SemaphoreType.DMA,
        ],
    )
    return k(ids, table)


def _mlp_body(x_ref, w1_ref, b1_ref, w2_ref, b2_ref, o_ref):
    x = x_ref[...]
    h = jnp.dot(x, w1_ref[...], preferred_element_type=jnp.float32)
    h = jnp.maximum(h + b1_ref[...], 0.0)
    o = jnp.dot(h, w2_ref[...], preferred_element_type=jnp.float32)
    o_ref[...] = o + b2_ref[...]


def _mlp_body_alias(x_ref, w1_ref, b1_ref, w2_ref, b2_ref, prev_ref, o_ref):
    del prev_ref
    _mlp_body(x_ref, w1_ref, b1_ref, w2_ref, b2_ref, o_ref)


_WSPECS = [
    pl.BlockSpec((D, D), lambda i: (0, 0)),
    pl.BlockSpec((1, D), lambda i: (0, 0)),
    pl.BlockSpec((D, D), lambda i: (0, 0)),
    pl.BlockSpec((1, D), lambda i: (0, 0)),
]


def _tc_mlp_part(x, w1, b1, w2, b2, n_tok, off_blk, prev, block_t=2048):
    nblk = x.shape[0] // block_t
    x_spec = pl.BlockSpec((block_t, D), lambda i: (i, 0))
    out_spec = pl.BlockSpec((block_t, D), lambda i: (i + off_blk, 0))
    if prev is None:
        return pl.pallas_call(
            _mlp_body,
            grid=(nblk,),
            in_specs=[x_spec] + _WSPECS,
            out_specs=out_spec,
            out_shape=jax.ShapeDtypeStruct((n_tok, D), jnp.float32),
        )(x, w1, b1.reshape(1, D), w2, b2.reshape(1, D))
    return pl.pallas_call(
        _mlp_body_alias,
        grid=(nblk,),
        in_specs=[x_spec] + _WSPECS + [pl.BlockSpec(memory_space=pl.ANY)],
        out_specs=out_spec,
        out_shape=jax.ShapeDtypeStruct((n_tok, D), jnp.float32),
        input_output_aliases={5: 0},
    )(x, w1, b1.reshape(1, D), w2, b2.reshape(1, D), prev)


CHUNK_SIZES = (16384, 16384)
BLOCK_T = 4096


def kernel(input_ids, emb_table, W1, b1, W2, b2):
    B, S = input_ids.shape
    ids = input_ids.reshape(-1).astype(jnp.int32)
    n_tok = ids.shape[0]
    offs = [sum(CHUNK_SIZES[:k]) for k in range(len(CHUNK_SIZES))]
    xs = [
        _sc_gather(ids, emb_table, offs[k], ct)
        for k, ct in enumerate(CHUNK_SIZES)
    ]
    out = None
    for k, ct in enumerate(CHUNK_SIZES):
        out = _tc_mlp_part(
            xs[k], W1, b1, W2, b2, n_tok,
            off_blk=offs[k] // BLOCK_T, prev=out, block_t=BLOCK_T,
        )
    return out.reshape(B, S, D)


# speedup vs baseline: 3.8629x; 1.0037x over previous
import functools

import jax
import jax.numpy as jnp
from jax import lax
from jax.experimental import pallas as pl
from jax.experimental.pallas import tpu as pltpu
from jax.experimental.pallas import tpu_sc as plsc

D = 512

_INFO = plsc.get_sparse_core_info()
NC = _INFO.num_cores
NS = _INFO.num_subcores
NW = NC * NS

CHUNK = 32
NBUF = 7


def _gather_body(n_chunks, offset, ids_hbm, table_hbm, out_hbm, idx_v, rows_v, sem_g, sem_s):
    b_per_w = n_chunks * CHUNK
    wid = lax.axis_index("s") * NC + lax.axis_index("c")
    base = wid * b_per_w
    pltpu.sync_copy(ids_hbm.at[pl.ds(offset + base, b_per_w)], idx_v)

    def start_gather(c):
        return pltpu.async_copy(
            table_hbm.at[idx_v.at[pl.ds(c * CHUNK, CHUNK)]],
            rows_v.at[c % NBUF],
            sem_g,
        )

    gat = {c: start_gather(c) for c in range(min(NBUF - 1, n_chunks))}
    sca = {}
    for c in range(n_chunks):
        gat.pop(c).wait()
        if c >= 1:
            sca.pop(c - 1).wait()
        if c + NBUF - 1 < n_chunks:
            gat[c + NBUF - 1] = start_gather(c + NBUF - 1)
        sca[c] = pltpu.async_copy(
            rows_v.at[c % NBUF],
            out_hbm.at[pl.ds(base + c * CHUNK, CHUNK)],
            sem_s,
        )
    sca.pop(n_chunks - 1).wait()


def _sc_gather(ids, table, offset, n_tok):
    b_per_w = n_tok // NW
    n_chunks = b_per_w // CHUNK
    mesh = plsc.VectorSubcoreMesh(core_axis_name="c", subcore_axis_name="s")
    k = pl.kernel(
        functools.partial(_gather_body, n_chunks, offset),
        out_type=jax.ShapeDtypeStruct((n_tok, D), jnp.float32),
        mesh=mesh,
        scratch_types=[
            pltpu.VMEM((n_chunks * CHUNK,), jnp.int32),
            pltpu.VMEM((NBUF, CHUNK, D), jnp.float32),
            pltpu.SemaphoreType.DMA,
            pltpu.SemaphoreType.DMA,
        ],
    )
    return k(ids, table)


def _mlp_body(x_ref, w1_ref, b1_ref, w2_ref, b2_ref, o_ref):
    x = x_ref[...]
    h = jnp.dot(x, w1_ref[...], preferred_element_type=jnp.float32)
    h = jnp.maximum(h + b1_ref[...], 0.0)
    o = jnp.dot(h, w2_ref[...], preferred_element_type=jnp.float32)
    o_ref[...] = o + b2_ref[...]


def _mlp_body_alias(x_ref, w1_ref, b1_ref, w2_ref, b2_ref, prev_ref, o_ref):
    del prev_ref
    _mlp_body(x_ref, w1_ref, b1_ref, w2_ref, b2_ref, o_ref)


_WSPECS = [
    pl.BlockSpec((D, D), lambda i: (0, 0)),
    pl.BlockSpec((1, D), lambda i: (0, 0)),
    pl.BlockSpec((D, D), lambda i: (0, 0)),
    pl.BlockSpec((1, D), lambda i: (0, 0)),
]


def _tc_mlp_part(x, w1, b1, w2, b2, n_tok, off_blk, prev, block_t=2048):
    nblk = x.shape[0] // block_t
    x_spec = pl.BlockSpec((block_t, D), lambda i: (i, 0))
    out_spec = pl.BlockSpec((block_t, D), lambda i: (i + off_blk, 0))
    if prev is None:
        return pl.pallas_call(
            _mlp_body,
            grid=(nblk,),
            in_specs=[x_spec] + _WSPECS,
            out_specs=out_spec,
            out_shape=jax.ShapeDtypeStruct((n_tok, D), jnp.float32),
        )(x, w1, b1.reshape(1, D), w2, b2.reshape(1, D))
    return pl.pallas_call(
        _mlp_body_alias,
        grid=(nblk,),
        in_specs=[x_spec] + _WSPECS + [pl.BlockSpec(memory_space=pl.ANY)],
        out_specs=out_spec,
        out_shape=jax.ShapeDtypeStruct((n_tok, D), jnp.float32),
        input_output_aliases={5: 0},
    )(x, w1, b1.reshape(1, D), w2, b2.reshape(1, D), prev)


CHUNK_SIZES = (16384, 16384)
BLOCK_T = 4096


def kernel(input_ids, emb_table, W1, b1, W2, b2):
    B, S = input_ids.shape
    ids = input_ids.reshape(-1).astype(jnp.int32)
    n_tok = ids.shape[0]
    offs = [sum(CHUNK_SIZES[:k]) for k in range(len(CHUNK_SIZES))]
    xs = [
        _sc_gather(ids, emb_table, offs[k], ct)
        for k, ct in enumerate(CHUNK_SIZES)
    ]
    out = None
    for k, ct in enumerate(CHUNK_SIZES):
        out = _tc_mlp_part(
            xs[k], W1, b1, W2, b2, n_tok,
            off_blk=offs[k] // BLOCK_T, prev=out, block_t=BLOCK_T,
        )
    return out.reshape(B, S, D)
